# baseline (speedup 1.0000x reference)
import functools

import jax
import jax.numpy as jnp
from jax import lax
from jax.experimental import pallas as pl
from jax.experimental.pallas import tpu as pltpu
from jax.experimental.pallas import tpu_sc as plsc

N = 10000
NP = 10240
E = 160000
H = 128
R = 4
K = 3
BN = 1024
GRID = NP // BN

NSC = 2
NTILE = 16
CH = 128
EPAD = 163840
EP_TILE = EPAD // (NSC * NTILE)
NCHUNK = EP_TILE // CH
NACC = 10112
ROWS_TILE = NACC // NTILE
_SPANS = [(0, 128), (128, 128), (256, 128), (384, 128), (512, 120)]


def _enc_body(idsb_ref, st_ref, ct_ref, ws_ref, bs_ref, w1a_ref, w1b_ref,
              b1_ref, w2_ref, b2_ref, wl_ref, bl_ref, h_ref, t_ref):
    f32 = jnp.float32
    dn = (((1,), (1,)), ((), ()))
    iota = lax.broadcasted_iota(jnp.int32, (BN, H), 1)
    oh = (idsb_ref[...] == iota).astype(f32)
    ce = jnp.dot(oh, ct_ref[...], preferred_element_type=f32)
    se = lax.dot_general(st_ref[...], ws_ref[...], dn,
                         preferred_element_type=f32) + bs_ref[...]
    ce = jnp.maximum(ce, 0.0)
    se = jnp.maximum(se, 0.0)
    x = (lax.dot_general(ce, w1a_ref[...], dn, preferred_element_type=f32)
         + lax.dot_general(se, w1b_ref[...], dn, preferred_element_type=f32)
         + b1_ref[...])
    x = jnp.maximum(x, 0.0)
    h = lax.dot_general(x, w2_ref[...], dn, preferred_element_type=f32) + b2_ref[...]
    h = jnp.maximum(h, 0.0)
    h_ref[...] = h
    t_ref[0] = jnp.zeros((BN, H), f32)
    bl = bl_ref[...]
    for i in range(R):
        t_ref[i + 1] = (lax.dot_general(h, wl_ref[i], dn,
                                        preferred_element_type=f32)
                        + bl[i:i + 1])


def _encoder(idsb, states_p, ct_p, ws_p, bs2, w1a, w1b, b12, w2, b22, wl, bl):
    full = lambda shp: pl.BlockSpec(shp, lambda i: tuple(0 for _ in shp))
    return pl.pallas_call(
        _enc_body,
        grid=(GRID,),
        in_specs=[
            pl.BlockSpec((BN, H), lambda i: (i, 0)),
            pl.BlockSpec((BN, H), lambda i: (i, 0)),
            full((H, 64)), full((64, H)), full((1, 64)),
            full((H, 64)), full((H, 64)), full((1, H)),
            full((H, H)), full((1, H)),
            full((R, H, H)), full((R, H)),
        ],
        out_specs=[
            pl.BlockSpec((BN, H), lambda i: (i, 0)),
            pl.BlockSpec((R + 1, BN, H), lambda i: (0, i, 0)),
        ],
        out_shape=[
            jax.ShapeDtypeStruct((NP, H), jnp.float32),
            jax.ShapeDtypeStruct((R + 1, NP, H), jnp.float32),
        ],
    )(idsb, states_p, ct_p, ws_p, bs2, w1a, w1b, b12, w2, b22, wl, bl)


def _gru_body(emit_t, ap_ref, h_ref, wih_ref, bih_ref, whh_ref, bhh_ref,
              wl_ref, bl_ref, hn_ref, *maybe_t):
    f32 = jnp.float32
    dn = (((1,), (1,)), ((), ()))
    a = ap_ref[0] + ap_ref[1]
    h = h_ref[...]
    gi = lax.dot_general(a, wih_ref[...], dn, preferred_element_type=f32) + bih_ref[...]
    gh = lax.dot_general(h, whh_ref[...], dn, preferred_element_type=f32) + bhh_ref[...]
    r = jax.nn.sigmoid(gi[:, 0:H] + gh[:, 0:H])
    z = jax.nn.sigmoid(gi[:, H:2 * H] + gh[:, H:2 * H])
    n = jnp.tanh(gi[:, 2 * H:3 * H] + r * gh[:, 2 * H:3 * H])
    hn = (1.0 - z) * n + z * h
    hn_ref[...] = hn
    if emit_t:
        t_ref = maybe_t[0]
        t_ref[0] = jnp.zeros((BN, H), f32)
        bl = bl_ref[...]
        for i in range(R):
            t_ref[i + 1] = (lax.dot_general(hn, wl_ref[i], dn,
                                            preferred_element_type=f32)
                            + bl[i:i + 1])


def _gru(emit_t, ap, h, wih, bih2, whh, bhh2, wl, bl):
    full = lambda shp: pl.BlockSpec(shp, lambda i: tuple(0 for _ in shp))
    out_specs = [pl.BlockSpec((BN, H), lambda i: (i, 0))]
    out_shape = [jax.ShapeDtypeStruct((NP, H), jnp.float32)]
    if emit_t:
        out_specs.append(pl.BlockSpec((R + 1, BN, H), lambda i: (0, i, 0)))
        out_shape.append(jax.ShapeDtypeStruct((R + 1, NP, H), jnp.float32))
    return pl.pallas_call(
        functools.partial(_gru_body, emit_t),
        grid=(GRID,),
        in_specs=[
            pl.BlockSpec((NSC, BN, H), lambda i: (0, i, 0)),
            pl.BlockSpec((BN, H), lambda i: (i, 0)),
            full((3 * H, H)), full((1, 3 * H)),
            full((3 * H, H)), full((1, 3 * H)),
            full((R, H, H)), full((R, H)),
        ],
        out_specs=out_specs,
        out_shape=out_shape,
    )(ap, h, wih, bih2, whh, bhh2, wl, bl)


@functools.cache
def _sc_kernel():
    return pl.kernel(
        _sc_body,
        mesh=plsc.VectorSubcoreMesh(core_axis_name="c", subcore_axis_name="s"),
        out_type=jax.ShapeDtypeStruct((NSC, NACC, H), jnp.float32),
        scratch_types=[
        pltpu.VMEM((CH,), jnp.int32),
        pltpu.VMEM((CH,), jnp.int32),
        pltpu.VMEM((CH,), jnp.int32),
        pltpu.VMEM((CH,), jnp.int32),
        pltpu.VMEM((CH,), jnp.int32),
        pltpu.VMEM((CH,), jnp.int32),
        pltpu.VMEM((CH,), jnp.int32),
        pltpu.VMEM((CH,), jnp.int32),
        pltpu.VMEM((CH, H), jnp.float32),
        pltpu.VMEM((CH, H), jnp.float32),
        pltpu.VMEM_SHARED((NACC, H), jnp.float32),
            pltpu.SemaphoreType.DMA,
            pltpu.SemaphoreType.DMA,
            pltpu.SemaphoreType.DMA,
            pltpu.SemaphoreType.DMA,
            pltpu.SemaphoreType.DMA,
        ],
    )


def _sc_body(t_hbm, src_hbm, et_hbm, dst_hbm, out_hbm,
             srcA, srcB, etA, etB, dstA, dstB, gidx0, gidx1,
             rows0, rows1, accum, semi, semg0, semg1, sems0, sems1):
    c = lax.axis_index("c")
    s = lax.axis_index("s")
    zeros16 = jnp.zeros((16,), jnp.float32)
    base_row = pl.multiple_of(s * ROWS_TILE, 8)
    tile_base = pl.multiple_of(c * (EPAD // NSC) + s * EP_TILE, CH)

    def _zrow(j, _):
        for k in range(H // 16):
            rows0[j, pl.ds(k * 16, 16)] = zeros16
        return 0
    lax.fori_loop(0, CH, _zrow, 0)
    for r0, ln in _SPANS:
        pltpu.sync_copy(rows0.at[pl.ds(0, ln)],
                        accum.at[pl.ds(base_row + r0, ln)])
    plsc.subcore_barrier()

    def _fill(et_c, src_c, gidx_c):
        def _gix(i, _):
            sl = pl.ds(pl.multiple_of(i * 16, 16), 16)
            gidx_c[sl] = et_c[sl] * NP + src_c[sl]
            return 0
        lax.fori_loop(0, CH // 16, _gix, 0)

    def _pair(m, _):
        off0 = pl.multiple_of(tile_base + m * (2 * CH), CH)
        off1 = pl.multiple_of(off0 + CH, CH)
        l0s = pltpu.async_copy(src_hbm.at[pl.ds(off0, CH)], srcA, semi)
        l0e = pltpu.async_copy(et_hbm.at[pl.ds(off0, CH)], etA, semi)
        l0d = pltpu.async_copy(dst_hbm.at[pl.ds(off0, CH)], dstA, semi)
        l1s = pltpu.async_copy(src_hbm.at[pl.ds(off1, CH)], srcB, semi)
        l1e = pltpu.async_copy(et_hbm.at[pl.ds(off1, CH)], etB, semi)
        l1d = pltpu.async_copy(dst_hbm.at[pl.ds(off1, CH)], dstB, semi)
        l0s.wait()
        l0e.wait()
        l0d.wait()
        l1s.wait()
        l1e.wait()
        l1d.wait()
        _fill(etA, srcA, gidx0)
        g0 = pltpu.async_copy(t_hbm.at[gidx0], rows0, semg0)
        _fill(etB, srcB, gidx1)
        g1 = pltpu.async_copy(t_hbm.at[gidx1], rows1, semg1)
        g0.wait()
        s0 = pltpu.async_copy(rows0, accum.at[dstA], sems0, add=True)
        g1.wait()
        s1 = pltpu.async_copy(rows1, accum.at[dstB], sems1, add=True)
        s0.wait()
        s1.wait()
        return 0
    lax.fori_loop(0, NCHUNK // 2, _pair, 0)
    plsc.subcore_barrier()

    for i, (r0, ln) in enumerate(_SPANS):
        buf = rows0 if i % 2 == 0 else rows1
        rr = base_row + r0
        pltpu.sync_copy(accum.at[pl.ds(rr, ln)], buf.at[pl.ds(0, ln)])
        pltpu.sync_copy(buf.at[pl.ds(0, ln)], out_hbm.at[c, pl.ds(rr, ln)])


def kernel(class_objects, states_objects, edge_tuples, edge_classes,
           mask_object, mask_edge, class_table, Ws, bs, W1, b1, W2, b2,
           Wl, bl, W_ih, b_ih, W_hh, b_hh):
    f32 = jnp.float32
    num_envs = class_objects.shape[0]

    ct_p = jnp.zeros((H, 64), f32).at[:class_table.shape[0]].set(class_table)
    ws_p = jnp.zeros((64, H), f32).at[:, :Ws.shape[1]].set(Ws)
    w1a = W1[:, :64]
    w1b = W1[:, 64:]
    bs2 = bs.reshape(1, 64)
    b12 = b1.reshape(1, H)
    b22 = b2.reshape(1, H)
    bih2 = b_ih.reshape(1, 3 * H)
    bhh2 = b_hh.reshape(1, 3 * H)

    epad = EPAD - E
    pad_dst = (jnp.arange(epad, dtype=jnp.int32) % N)

    outs = []
    for env in range(num_envs):
        ids = class_objects[env].astype(jnp.int32)
        ids_p = jnp.zeros((NP,), jnp.int32).at[:N].set(ids)
        idsb = jnp.broadcast_to(ids_p[:, None], (NP, H))
        states_p = jnp.zeros((NP, H), f32).at[:N, :states_objects.shape[2]].set(
            states_objects[env])

        src = edge_tuples[env, :, 0].astype(jnp.int32)
        dst = edge_tuples[env, :, 1].astype(jnp.int32)
        et = edge_classes[env].astype(jnp.int32)
        src_p = jnp.concatenate([src, jnp.zeros((epad,), jnp.int32)])
        et_p = jnp.concatenate([et, jnp.zeros((epad,), jnp.int32)])
        dst_p = jnp.concatenate([dst, pad_dst])
        order = jnp.argsort(et_p * NP + src_p)
        src_p = src_p[order]
        et_p = et_p[order]
        dst_p = dst_p[order]

        h, t = _encoder(idsb, states_p, ct_p, ws_p, bs2, w1a, w1b, b12,
                        W2, b22, Wl, bl)
        for step in range(K):
            ap = _sc_kernel()(t.reshape((R + 1) * NP, H), src_p, et_p, dst_p)
            ap = jnp.pad(ap, ((0, 0), (0, NP - NACC), (0, 0)))
            if step < K - 1:
                h, t = _gru(True, ap, h, W_ih, bih2, W_hh, bhh2, Wl, bl)
            else:
                (h,) = _gru(False, ap, h, W_ih, bih2, W_hh, bhh2, Wl, bl)
        outs.append(h[:N])
    return jnp.stack(outs, axis=0)


# speedup vs baseline: 1.4952x; 1.4952x over previous
import functools

import jax
import jax.numpy as jnp
from jax import lax
from jax.experimental import pallas as pl
from jax.experimental.pallas import tpu as pltpu
from jax.experimental.pallas import tpu_sc as plsc

N = 10000
NP = 10240
E = 160000
H = 128
R = 4
K = 3
BN = 1024
GRID = NP // BN

NSC = 2
NTILE = 16
CH = 64
NBUF = 4
QCH = CH * NBUF
EPAD = 163840
EP_TILE = EPAD // (NSC * NTILE)
NQUAD = EP_TILE // QCH
NACC = 10112
ROWS_TILE = NACC // NTILE
_SPANS = [(0, 128), (128, 128), (256, 128), (384, 128), (512, 120)]


def _enc_body(idsb_ref, st_ref, ct_ref, ws_ref, bs_ref, w1a_ref, w1b_ref,
              b1_ref, w2_ref, b2_ref, wl_ref, bl_ref, h_ref, t_ref):
    f32 = jnp.float32
    dn = (((1,), (1,)), ((), ()))
    iota = lax.broadcasted_iota(jnp.int32, (BN, H), 1)
    oh = (idsb_ref[...] == iota).astype(f32)
    ce = jnp.dot(oh, ct_ref[...], preferred_element_type=f32)
    se = lax.dot_general(st_ref[...], ws_ref[...], dn,
                         preferred_element_type=f32) + bs_ref[...]
    ce = jnp.maximum(ce, 0.0)
    se = jnp.maximum(se, 0.0)
    x = (lax.dot_general(ce, w1a_ref[...], dn, preferred_element_type=f32)
         + lax.dot_general(se, w1b_ref[...], dn, preferred_element_type=f32)
         + b1_ref[...])
    x = jnp.maximum(x, 0.0)
    h = lax.dot_general(x, w2_ref[...], dn, preferred_element_type=f32) + b2_ref[...]
    h = jnp.maximum(h, 0.0)
    h_ref[...] = h
    t_ref[0] = jnp.zeros((BN, H), f32)
    bl = bl_ref[...]
    for i in range(R):
        t_ref[i + 1] = (lax.dot_general(h, wl_ref[i], dn,
                                        preferred_element_type=f32)
                        + bl[i:i + 1])


def _encoder(idsb, states_p, ct_p, ws_p, bs2, w1a, w1b, b12, w2, b22, wl, bl):
    full = lambda shp: pl.BlockSpec(shp, lambda i: tuple(0 for _ in shp))
    return pl.pallas_call(
        _enc_body,
        grid=(GRID,),
        in_specs=[
            pl.BlockSpec((BN, H), lambda i: (i, 0)),
            pl.BlockSpec((BN, H), lambda i: (i, 0)),
            full((H, 64)), full((64, H)), full((1, 64)),
            full((H, 64)), full((H, 64)), full((1, H)),
            full((H, H)), full((1, H)),
            full((R, H, H)), full((R, H)),
        ],
        out_specs=[
            pl.BlockSpec((BN, H), lambda i: (i, 0)),
            pl.BlockSpec((R + 1, BN, H), lambda i: (0, i, 0)),
        ],
        out_shape=[
            jax.ShapeDtypeStruct((NP, H), jnp.float32),
            jax.ShapeDtypeStruct((R + 1, NP, H), jnp.float32),
        ],
    )(idsb, states_p, ct_p, ws_p, bs2, w1a, w1b, b12, w2, b22, wl, bl)


def _gru_body(emit_t, ap_ref, h_ref, wih_ref, bih_ref, whh_ref, bhh_ref,
              wl_ref, bl_ref, hn_ref, *maybe_t):
    f32 = jnp.float32
    dn = (((1,), (1,)), ((), ()))
    a = ap_ref[0] + ap_ref[1]
    h = h_ref[...]
    gi = lax.dot_general(a, wih_ref[...], dn, preferred_element_type=f32) + bih_ref[...]
    gh = lax.dot_general(h, whh_ref[...], dn, preferred_element_type=f32) + bhh_ref[...]
    r = jax.nn.sigmoid(gi[:, 0:H] + gh[:, 0:H])
    z = jax.nn.sigmoid(gi[:, H:2 * H] + gh[:, H:2 * H])
    n = jnp.tanh(gi[:, 2 * H:3 * H] + r * gh[:, 2 * H:3 * H])
    hn = (1.0 - z) * n + z * h
    hn_ref[...] = hn
    if emit_t:
        t_ref = maybe_t[0]
        t_ref[0] = jnp.zeros((BN, H), f32)
        bl = bl_ref[...]
        for i in range(R):
            t_ref[i + 1] = (lax.dot_general(hn, wl_ref[i], dn,
                                            preferred_element_type=f32)
                            + bl[i:i + 1])


def _gru(emit_t, ap, h, wih, bih2, whh, bhh2, wl, bl):
    full = lambda shp: pl.BlockSpec(shp, lambda i: tuple(0 for _ in shp))
    out_specs = [pl.BlockSpec((BN, H), lambda i: (i, 0))]
    out_shape = [jax.ShapeDtypeStruct((NP, H), jnp.float32)]
    if emit_t:
        out_specs.append(pl.BlockSpec((R + 1, BN, H), lambda i: (0, i, 0)))
        out_shape.append(jax.ShapeDtypeStruct((R + 1, NP, H), jnp.float32))
    return pl.pallas_call(
        functools.partial(_gru_body, emit_t),
        grid=(GRID,),
        in_specs=[
            pl.BlockSpec((NSC, BN, H), lambda i: (0, i, 0)),
            pl.BlockSpec((BN, H), lambda i: (i, 0)),
            full((3 * H, H)), full((1, 3 * H)),
            full((3 * H, H)), full((1, 3 * H)),
            full((R, H, H)), full((R, H)),
        ],
        out_specs=out_specs,
        out_shape=out_shape,
    )(ap, h, wih, bih2, whh, bhh2, wl, bl)


@functools.cache
def _sc_kernel():
    return pl.kernel(
        _sc_body,
        mesh=plsc.VectorSubcoreMesh(core_axis_name="c", subcore_axis_name="s"),
        out_type=jax.ShapeDtypeStruct((NSC, NACC, H), jnp.float32),
        scratch_types=[
            pltpu.VMEM((QCH,), jnp.int32),
            pltpu.VMEM((QCH,), jnp.int32),
            pltpu.VMEM((QCH,), jnp.int32),
            pltpu.VMEM((CH,), jnp.int32),
            pltpu.VMEM((CH,), jnp.int32),
            pltpu.VMEM((CH,), jnp.int32),
            pltpu.VMEM((CH,), jnp.int32),
            pltpu.VMEM((CH,), jnp.int32),
            pltpu.VMEM((CH,), jnp.int32),
            pltpu.VMEM((CH,), jnp.int32),
            pltpu.VMEM((CH,), jnp.int32),
            pltpu.VMEM((CH, H), jnp.float32),
            pltpu.VMEM((CH, H), jnp.float32),
            pltpu.VMEM((CH, H), jnp.float32),
            pltpu.VMEM((CH, H), jnp.float32),
            pltpu.VMEM_SHARED((NACC, H), jnp.float32),
            pltpu.SemaphoreType.DMA,
            pltpu.SemaphoreType.DMA,
            pltpu.SemaphoreType.DMA,
            pltpu.SemaphoreType.DMA,
            pltpu.SemaphoreType.DMA,
            pltpu.SemaphoreType.DMA,
            pltpu.SemaphoreType.DMA,
            pltpu.SemaphoreType.DMA,
            pltpu.SemaphoreType.DMA,
        ],
    )


def _sc_body(t_hbm, src_hbm, et_hbm, dst_hbm, out_hbm,
             src_q, et_q, dst_q, gi0, gi1, gi2, gi3, dc0, dc1, dc2, dc3,
             rw0, rw1, rw2, rw3, accum, semi,
             sg0, sg1, sg2, sg3, ss0, ss1, ss2, ss3):
    gidx = (gi0, gi1, gi2, gi3)
    dstc = (dc0, dc1, dc2, dc3)
    rows = (rw0, rw1, rw2, rw3)
    semg = (sg0, sg1, sg2, sg3)
    sems = (ss0, ss1, ss2, ss3)
    c = lax.axis_index("c")
    s = lax.axis_index("s")
    zeros16 = jnp.zeros((16,), jnp.float32)
    base_row = pl.multiple_of(s * ROWS_TILE, 8)
    tile_base = pl.multiple_of(c * (EPAD // NSC) + s * EP_TILE, QCH)

    def _zrow(j, _):
        for k in range(H // 16):
            rw0[j, pl.ds(k * 16, 16)] = zeros16
        return 0
    lax.fori_loop(0, CH, _zrow, 0)
    zspans = [(k * CH, CH) for k in range(ROWS_TILE // CH)]
    zspans.append((ROWS_TILE // CH * CH, ROWS_TILE % CH))
    for r0, ln in zspans:
        pltpu.sync_copy(rw0.at[pl.ds(0, ln)],
                        accum.at[pl.ds(base_row + r0, ln)])
    plsc.subcore_barrier()

    def _quad(m, _):
        off = pl.multiple_of(tile_base + m * QCH, QCH)
        lq_s = pltpu.async_copy(src_hbm.at[pl.ds(off, QCH)], src_q, semi)
        lq_e = pltpu.async_copy(et_hbm.at[pl.ds(off, QCH)], et_q, semi)
        lq_d = pltpu.async_copy(dst_hbm.at[pl.ds(off, QCH)], dst_q, semi)
        lq_s.wait()
        lq_e.wait()
        lq_d.wait()
        gs = []
        for j in range(NBUF):
            def _gix(i, _, j=j):
                sl = pl.ds(pl.multiple_of(i * 16, 16), 16)
                qsl = pl.ds(pl.multiple_of(j * CH + i * 16, 16), 16)
                gidx[j][sl] = et_q[qsl] * NP + src_q[qsl]
                dstc[j][sl] = dst_q[qsl]
                return 0
            lax.fori_loop(0, CH // 16, _gix, 0)
            gs.append(pltpu.async_copy(t_hbm.at[gidx[j]], rows[j], semg[j]))
        sc = []
        for j in range(NBUF):
            gs[j].wait()
            sc.append(pltpu.async_copy(rows[j], accum.at[dstc[j]],
                                       sems[j], add=True))
        for j in range(NBUF):
            sc[j].wait()
        return 0
    lax.fori_loop(0, NQUAD, _quad, 0)
    plsc.subcore_barrier()

    for i, (r0, ln) in enumerate(zspans):
        buf = rows[i % NBUF]
        rr = base_row + r0
        pltpu.sync_copy(accum.at[pl.ds(rr, ln)], buf.at[pl.ds(0, ln)])
        pltpu.sync_copy(buf.at[pl.ds(0, ln)], out_hbm.at[c, pl.ds(rr, ln)])


def kernel(class_objects, states_objects, edge_tuples, edge_classes,
           mask_object, mask_edge, class_table, Ws, bs, W1, b1, W2, b2,
           Wl, bl, W_ih, b_ih, W_hh, b_hh):
    f32 = jnp.float32
    num_envs = class_objects.shape[0]

    ct_p = jnp.zeros((H, 64), f32).at[:class_table.shape[0]].set(class_table)
    ws_p = jnp.zeros((64, H), f32).at[:, :Ws.shape[1]].set(Ws)
    w1a = W1[:, :64]
    w1b = W1[:, 64:]
    bs2 = bs.reshape(1, 64)
    b12 = b1.reshape(1, H)
    b22 = b2.reshape(1, H)
    bih2 = b_ih.reshape(1, 3 * H)
    bhh2 = b_hh.reshape(1, 3 * H)

    epad = EPAD - E
    pad_dst = (jnp.arange(epad, dtype=jnp.int32) % N)

    outs = []
    for env in range(num_envs):
        ids = class_objects[env].astype(jnp.int32)
        ids_p = jnp.zeros((NP,), jnp.int32).at[:N].set(ids)
        idsb = jnp.broadcast_to(ids_p[:, None], (NP, H))
        states_p = jnp.zeros((NP, H), f32).at[:N, :states_objects.shape[2]].set(
            states_objects[env])

        src = edge_tuples[env, :, 0].astype(jnp.int32)
        dst = edge_tuples[env, :, 1].astype(jnp.int32)
        et = edge_classes[env].astype(jnp.int32)
        src_p = jnp.concatenate([src, jnp.zeros((epad,), jnp.int32)])
        et_p = jnp.concatenate([et, jnp.zeros((epad,), jnp.int32)])
        dst_p = jnp.concatenate([dst, pad_dst])

        h, t = _encoder(idsb, states_p, ct_p, ws_p, bs2, w1a, w1b, b12,
                        W2, b22, Wl, bl)
        for step in range(K):
            ap = _sc_kernel()(t.reshape((R + 1) * NP, H), src_p, et_p, dst_p)
            ap = jnp.pad(ap, ((0, 0), (0, NP - NACC), (0, 0)))
            if step < K - 1:
                h, t = _gru(True, ap, h, W_ih, bih2, W_hh, bhh2, Wl, bl)
            else:
                (h,) = _gru(False, ap, h, W_ih, bih2, W_hh, bhh2, Wl, bl)
        outs.append(h[:N])
    return jnp.stack(outs, axis=0)


# speedup vs baseline: 1.4967x; 1.0010x over previous
import functools

import jax
import jax.numpy as jnp
from jax import lax
from jax.experimental import pallas as pl
from jax.experimental.pallas import tpu as pltpu
from jax.experimental.pallas import tpu_sc as plsc

N = 10000
NP = 10240
E = 160000
H = 128
R = 4
K = 3
BN = 1024
GRID = NP // BN

NSC = 2
NTILE = 16
CH = 64
NBUF = 4
QCH = CH * NBUF
EPAD = 163840
EP_TILE = EPAD // (NSC * NTILE)
NQUAD = EP_TILE // QCH
NACC = 10112
ROWS_TILE = NACC // NTILE
_SPANS = [(0, 128), (128, 128), (256, 128), (384, 128), (512, 120)]


def _enc_body(idsb_ref, st_ref, ct_ref, ws_ref, bs_ref, w1a_ref, w1b_ref,
              b1_ref, w2_ref, b2_ref, wl_ref, bl_ref, h_ref, t_ref):
    f32 = jnp.float32
    dn = (((1,), (1,)), ((), ()))
    iota = lax.broadcasted_iota(jnp.int32, (BN, H), 1)
    oh = (idsb_ref[...] == iota).astype(f32)
    ce = jnp.dot(oh, ct_ref[...], preferred_element_type=f32)
    se = lax.dot_general(st_ref[...], ws_ref[...], dn,
                         preferred_element_type=f32) + bs_ref[...]
    ce = jnp.maximum(ce, 0.0)
    se = jnp.maximum(se, 0.0)
    x = (lax.dot_general(ce, w1a_ref[...], dn, preferred_element_type=f32)
         + lax.dot_general(se, w1b_ref[...], dn, preferred_element_type=f32)
         + b1_ref[...])
    x = jnp.maximum(x, 0.0)
    h = lax.dot_general(x, w2_ref[...], dn, preferred_element_type=f32) + b2_ref[...]
    h = jnp.maximum(h, 0.0)
    h_ref[...] = h
    t_ref[0] = jnp.zeros((BN, H), f32)
    bl = bl_ref[...]
    for i in range(R):
        t_ref[i + 1] = (lax.dot_general(h, wl_ref[i], dn,
                                        preferred_element_type=f32)
                        + bl[i:i + 1])


def _encoder(idsb, states_p, ct_p, ws_p, bs2, w1a, w1b, b12, w2, b22, wl, bl):
    full = lambda shp: pl.BlockSpec(shp, lambda i: tuple(0 for _ in shp))
    return pl.pallas_call(
        _enc_body,
        grid=(GRID,),
        in_specs=[
            pl.BlockSpec((BN, H), lambda i: (i, 0)),
            pl.BlockSpec((BN, H), lambda i: (i, 0)),
            full((H, 64)), full((64, H)), full((1, 64)),
            full((H, 64)), full((H, 64)), full((1, H)),
            full((H, H)), full((1, H)),
            full((R, H, H)), full((R, H)),
        ],
        out_specs=[
            pl.BlockSpec((BN, H), lambda i: (i, 0)),
            pl.BlockSpec((R + 1, BN, H), lambda i: (0, i, 0)),
        ],
        out_shape=[
            jax.ShapeDtypeStruct((NP, H), jnp.float32),
            jax.ShapeDtypeStruct((R + 1, NP, H), jnp.float32),
        ],
    )(idsb, states_p, ct_p, ws_p, bs2, w1a, w1b, b12, w2, b22, wl, bl)


def _gru_body(emit_t, ap_ref, h_ref, wih_ref, bih_ref, whh_ref, bhh_ref,
              wl_ref, bl_ref, hn_ref, *maybe_t):
    f32 = jnp.float32
    dn = (((1,), (1,)), ((), ()))
    a = ap_ref[0] + ap_ref[1]
    h = h_ref[...]
    gi = lax.dot_general(a, wih_ref[...], dn, preferred_element_type=f32) + bih_ref[...]
    gh = lax.dot_general(h, whh_ref[...], dn, preferred_element_type=f32) + bhh_ref[...]
    r = jax.nn.sigmoid(gi[:, 0:H] + gh[:, 0:H])
    z = jax.nn.sigmoid(gi[:, H:2 * H] + gh[:, H:2 * H])
    n = jnp.tanh(gi[:, 2 * H:3 * H] + r * gh[:, 2 * H:3 * H])
    hn = (1.0 - z) * n + z * h
    hn_ref[...] = hn
    if emit_t:
        t_ref = maybe_t[0]
        t_ref[0] = jnp.zeros((BN, H), f32)
        bl = bl_ref[...]
        for i in range(R):
            t_ref[i + 1] = (lax.dot_general(hn, wl_ref[i], dn,
                                            preferred_element_type=f32)
                            + bl[i:i + 1])


def _gru(emit_t, ap, h, wih, bih2, whh, bhh2, wl, bl):
    full = lambda shp: pl.BlockSpec(shp, lambda i: tuple(0 for _ in shp))
    out_specs = [pl.BlockSpec((BN, H), lambda i: (i, 0))]
    out_shape = [jax.ShapeDtypeStruct((NP, H), jnp.float32)]
    if emit_t:
        out_specs.append(pl.BlockSpec((R + 1, BN, H), lambda i: (0, i, 0)))
        out_shape.append(jax.ShapeDtypeStruct((R + 1, NP, H), jnp.float32))
    return pl.pallas_call(
        functools.partial(_gru_body, emit_t),
        grid=(GRID,),
        in_specs=[
            pl.BlockSpec((NSC, BN, H), lambda i: (0, i, 0)),
            pl.BlockSpec((BN, H), lambda i: (i, 0)),
            full((3 * H, H)), full((1, 3 * H)),
            full((3 * H, H)), full((1, 3 * H)),
            full((R, H, H)), full((R, H)),
        ],
        out_specs=out_specs,
        out_shape=out_shape,
    )(ap, h, wih, bih2, whh, bhh2, wl, bl)


@functools.cache
def _sc_kernel():
    return pl.kernel(
        _sc_body,
        mesh=plsc.VectorSubcoreMesh(core_axis_name="c", subcore_axis_name="s"),
        out_type=jax.ShapeDtypeStruct((NSC, NACC, H), jnp.float32),
        scratch_types=[
            pltpu.VMEM((QCH,), jnp.int32),
            pltpu.VMEM((QCH,), jnp.int32),
            pltpu.VMEM((QCH,), jnp.int32),
            pltpu.VMEM((CH,), jnp.int32),
            pltpu.VMEM((CH,), jnp.int32),
            pltpu.VMEM((CH,), jnp.int32),
            pltpu.VMEM((CH,), jnp.int32),
            pltpu.VMEM((CH,), jnp.int32),
            pltpu.VMEM((CH,), jnp.int32),
            pltpu.VMEM((CH,), jnp.int32),
            pltpu.VMEM((CH,), jnp.int32),
            pltpu.VMEM((CH, H), jnp.float32),
            pltpu.VMEM((CH, H), jnp.float32),
            pltpu.VMEM((CH, H), jnp.float32),
            pltpu.VMEM((CH, H), jnp.float32),
            pltpu.VMEM_SHARED((NACC, H), jnp.float32),
            pltpu.SemaphoreType.DMA,
            pltpu.SemaphoreType.DMA,
            pltpu.SemaphoreType.DMA,
            pltpu.SemaphoreType.DMA,
            pltpu.SemaphoreType.DMA,
            pltpu.SemaphoreType.DMA,
            pltpu.SemaphoreType.DMA,
            pltpu.SemaphoreType.DMA,
            pltpu.SemaphoreType.DMA,
        ],
    )


def _sc_body(t_hbm, src_hbm, et_hbm, dst_hbm, out_hbm,
             src_q, et_q, dst_q, gi0, gi1, gi2, gi3, dc0, dc1, dc2, dc3,
             rw0, rw1, rw2, rw3, accum, semi,
             sg0, sg1, sg2, sg3, ss0, ss1, ss2, ss3):
    gidx = (gi0, gi1, gi2, gi3)
    dstc = (dc0, dc1, dc2, dc3)
    rows = (rw0, rw1, rw2, rw3)
    semg = (sg0, sg1, sg2, sg3)
    sems = (ss0, ss1, ss2, ss3)
    c = lax.axis_index("c")
    s = lax.axis_index("s")
    zeros16 = jnp.zeros((16,), jnp.float32)
    base_row = pl.multiple_of(s * ROWS_TILE, 8)
    tile_base = pl.multiple_of(c * (EPAD // NSC) + s * EP_TILE, QCH)

    def _zrow(j, _):
        for k in range(H // 16):
            rw0[j, pl.ds(k * 16, 16)] = zeros16
        return 0
    lax.fori_loop(0, CH, _zrow, 0)
    zspans = [(k * CH, CH) for k in range(ROWS_TILE // CH)]
    zspans.append((ROWS_TILE // CH * CH, ROWS_TILE % CH))
    for r0, ln in zspans:
        pltpu.sync_copy(rw0.at[pl.ds(0, ln)],
                        accum.at[pl.ds(base_row + r0, ln)])
    plsc.subcore_barrier()

    def _quad(m, _):
        off = pl.multiple_of(tile_base + m * QCH, QCH)
        lq_s = pltpu.async_copy(src_hbm.at[pl.ds(off, QCH)], src_q, semi)
        lq_e = pltpu.async_copy(et_hbm.at[pl.ds(off, QCH)], et_q, semi)
        lq_d = pltpu.async_copy(dst_hbm.at[pl.ds(off, QCH)], dst_q, semi)
        lq_s.wait()
        lq_e.wait()
        lq_d.wait()
        gs = []
        for j in range(NBUF):
            def _gix(i, _, j=j):
                sl = pl.ds(pl.multiple_of(i * 16, 16), 16)
                qsl = pl.ds(pl.multiple_of(j * CH + i * 16, 16), 16)
                gidx[j][sl] = et_q[qsl] * NP + src_q[qsl]
                dstc[j][sl] = dst_q[qsl]
                return 0
            lax.fori_loop(0, CH // 16, _gix, 0)
            gs.append(pltpu.async_copy(t_hbm.at[gidx[j]], rows[j], semg[j]))
        sc = []
        for j in range(NBUF):
            gs[j].wait()
            sc.append(pltpu.async_copy(rows[j], accum.at[dstc[j]],
                                       sems[j], add=True))
        for j in range(NBUF):
            sc[j].wait()
        return 0
    lax.fori_loop(0, NQUAD, _quad, 0)
    plsc.subcore_barrier()

    for i, (r0, ln) in enumerate(zspans):
        buf = rows[i % NBUF]
        rr = base_row + r0
        pltpu.sync_copy(accum.at[pl.ds(rr, ln)], buf.at[pl.ds(0, ln)])
        pltpu.sync_copy(buf.at[pl.ds(0, ln)], out_hbm.at[c, pl.ds(rr, ln)])


def kernel(class_objects, states_objects, edge_tuples, edge_classes,
           mask_object, mask_edge, class_table, Ws, bs, W1, b1, W2, b2,
           Wl, bl, W_ih, b_ih, W_hh, b_hh):
    f32 = jnp.float32
    num_envs = class_objects.shape[0]

    ct_p = jnp.zeros((H, 64), f32).at[:class_table.shape[0]].set(class_table)
    ws_p = jnp.zeros((64, H), f32).at[:, :Ws.shape[1]].set(Ws)
    w1a = W1[:, :64]
    w1b = W1[:, 64:]
    bs2 = bs.reshape(1, 64)
    b12 = b1.reshape(1, H)
    b22 = b2.reshape(1, H)
    bih2 = b_ih.reshape(1, 3 * H)
    bhh2 = b_hh.reshape(1, 3 * H)

    epad = EPAD - E
    pad_dst = (jnp.arange(epad, dtype=jnp.int32) % N)

    hs, ts, edges = [], [], []
    for env in range(num_envs):
        ids = class_objects[env].astype(jnp.int32)
        ids_p = jnp.zeros((NP,), jnp.int32).at[:N].set(ids)
        idsb = jnp.broadcast_to(ids_p[:, None], (NP, H))
        states_p = jnp.zeros((NP, H), f32).at[:N, :states_objects.shape[2]].set(
            states_objects[env])

        src = edge_tuples[env, :, 0].astype(jnp.int32)
        dst = edge_tuples[env, :, 1].astype(jnp.int32)
        et = edge_classes[env].astype(jnp.int32)
        edges.append((
            jnp.concatenate([src, jnp.zeros((epad,), jnp.int32)]),
            jnp.concatenate([et, jnp.zeros((epad,), jnp.int32)]),
            jnp.concatenate([dst, pad_dst]),
        ))
        h, t = _encoder(idsb, states_p, ct_p, ws_p, bs2, w1a, w1b, b12,
                        W2, b22, Wl, bl)
        hs.append(h)
        ts.append(t)

    for step in range(K):
        aps = []
        for env in range(num_envs):
            src_p, et_p, dst_p = edges[env]
            ap = _sc_kernel()(ts[env].reshape((R + 1) * NP, H),
                              src_p, et_p, dst_p)
            aps.append(jnp.pad(ap, ((0, 0), (0, NP - NACC), (0, 0))))
        for env in range(num_envs):
            if step < K - 1:
                hs[env], ts[env] = _gru(True, aps[env], hs[env], W_ih, bih2,
                                        W_hh, bhh2, Wl, bl)
            else:
                (hs[env],) = _gru(False, aps[env], hs[env], W_ih, bih2,
                                  W_hh, bhh2, Wl, bl)
    return jnp.stack([h[:N] for h in hs], axis=0)


# speedup vs baseline: 1.5993x; 1.0686x over previous
import functools

import jax
import jax.numpy as jnp
from jax import lax
from jax.experimental import pallas as pl
from jax.experimental.pallas import tpu as pltpu
from jax.experimental.pallas import tpu_sc as plsc

N = 10000
NP = 10240
E = 160000
H = 128
R = 4
K = 3
BN = 1024
GRID = NP // BN

NSC = 2
NTILE = 16
CH = 64
NBUF = 4
QCH = CH * NBUF
EPAD = 163840
EP_TILE = EPAD // (NSC * NTILE)
NQUAD = EP_TILE // QCH
Q0 = 28
Q1 = 2 * NQUAD - Q0
NACC = 10112
ROWS_TILE = NACC // NTILE
_SPANS = [(0, 128), (128, 128), (256, 128), (384, 128), (512, 120)]


def _enc_body(idsb_ref, st_ref, ct_ref, ws_ref, bs_ref, w1a_ref, w1b_ref,
              b1_ref, w2_ref, b2_ref, wl_ref, bl_ref, h_ref, t_ref):
    f32 = jnp.float32
    dn = (((1,), (1,)), ((), ()))
    iota = lax.broadcasted_iota(jnp.int32, (BN, H), 1)
    oh = (idsb_ref[...] == iota).astype(f32)
    ce = jnp.dot(oh, ct_ref[...], preferred_element_type=f32)
    se = lax.dot_general(st_ref[...], ws_ref[...], dn,
                         preferred_element_type=f32) + bs_ref[...]
    ce = jnp.maximum(ce, 0.0)
    se = jnp.maximum(se, 0.0)
    x = (lax.dot_general(ce, w1a_ref[...], dn, preferred_element_type=f32)
         + lax.dot_general(se, w1b_ref[...], dn, preferred_element_type=f32)
         + b1_ref[...])
    x = jnp.maximum(x, 0.0)
    h = lax.dot_general(x, w2_ref[...], dn, preferred_element_type=f32) + b2_ref[...]
    h = jnp.maximum(h, 0.0)
    h_ref[...] = h
    t_ref[0] = jnp.zeros((BN, H), f32)
    bl = bl_ref[...]
    for i in range(R):
        t_ref[i + 1] = (lax.dot_general(h, wl_ref[i], dn,
                                        preferred_element_type=f32)
                        + bl[i:i + 1])


def _encoder(idsb, states_p, ct_p, ws_p, bs2, w1a, w1b, b12, w2, b22, wl, bl):
    full = lambda shp: pl.BlockSpec(shp, lambda i: tuple(0 for _ in shp))
    return pl.pallas_call(
        _enc_body,
        grid=(GRID,),
        in_specs=[
            pl.BlockSpec((BN, H), lambda i: (i, 0)),
            pl.BlockSpec((BN, H), lambda i: (i, 0)),
            full((H, 64)), full((64, H)), full((1, 64)),
            full((H, 64)), full((H, 64)), full((1, H)),
            full((H, H)), full((1, H)),
            full((R, H, H)), full((R, H)),
        ],
        out_specs=[
            pl.BlockSpec((BN, H), lambda i: (i, 0)),
            pl.BlockSpec((R + 1, BN, H), lambda i: (0, i, 0)),
        ],
        out_shape=[
            jax.ShapeDtypeStruct((NP, H), jnp.float32),
            jax.ShapeDtypeStruct((R + 1, NP, H), jnp.float32),
        ],
    )(idsb, states_p, ct_p, ws_p, bs2, w1a, w1b, b12, w2, b22, wl, bl)


def _gru_body(emit_t, ap_ref, h_ref, wih_ref, bih_ref, whh_ref, bhh_ref,
              wl_ref, bl_ref, hn_ref, *maybe_t):
    f32 = jnp.float32
    dn = (((1,), (1,)), ((), ()))
    a = ap_ref[0] + ap_ref[1]
    h = h_ref[...]
    gi = lax.dot_general(a, wih_ref[...], dn, preferred_element_type=f32) + bih_ref[...]
    gh = lax.dot_general(h, whh_ref[...], dn, preferred_element_type=f32) + bhh_ref[...]
    r = jax.nn.sigmoid(gi[:, 0:H] + gh[:, 0:H])
    z = jax.nn.sigmoid(gi[:, H:2 * H] + gh[:, H:2 * H])
    n = jnp.tanh(gi[:, 2 * H:3 * H] + r * gh[:, 2 * H:3 * H])
    hn = (1.0 - z) * n + z * h
    hn_ref[...] = hn
    if emit_t:
        t_ref = maybe_t[0]
        t_ref[0] = jnp.zeros((BN, H), f32)
        bl = bl_ref[...]
        for i in range(R):
            t_ref[i + 1] = (lax.dot_general(hn, wl_ref[i], dn,
                                            preferred_element_type=f32)
                            + bl[i:i + 1])


def _gru(emit_t, ap, h, wih, bih2, whh, bhh2, wl, bl):
    full = lambda shp: pl.BlockSpec(shp, lambda i: tuple(0 for _ in shp))
    out_specs = [pl.BlockSpec((BN, H), lambda i: (i, 0))]
    out_shape = [jax.ShapeDtypeStruct((NP, H), jnp.float32)]
    if emit_t:
        out_specs.append(pl.BlockSpec((R + 1, BN, H), lambda i: (0, i, 0)))
        out_shape.append(jax.ShapeDtypeStruct((R + 1, NP, H), jnp.float32))
    return pl.pallas_call(
        functools.partial(_gru_body, emit_t),
        grid=(GRID,),
        in_specs=[
            pl.BlockSpec((NSC, BN, H), lambda i: (0, i, 0)),
            pl.BlockSpec((BN, H), lambda i: (i, 0)),
            full((3 * H, H)), full((1, 3 * H)),
            full((3 * H, H)), full((1, 3 * H)),
            full((R, H, H)), full((R, H)),
        ],
        out_specs=out_specs,
        out_shape=out_shape,
    )(ap, h, wih, bih2, whh, bhh2, wl, bl)


@functools.cache
def _sc_kernel():
    return pl.kernel(
        _sc_body,
        mesh=plsc.VectorSubcoreMesh(core_axis_name="c", subcore_axis_name="s"),
        out_type=jax.ShapeDtypeStruct((NSC, NACC, H), jnp.float32),
        scratch_types=[
            pltpu.VMEM((QCH,), jnp.int32),
            pltpu.VMEM((QCH,), jnp.int32),
            pltpu.VMEM((QCH,), jnp.int32),
            pltpu.VMEM((CH,), jnp.int32),
            pltpu.VMEM((CH,), jnp.int32),
            pltpu.VMEM((CH,), jnp.int32),
            pltpu.VMEM((CH,), jnp.int32),
            pltpu.VMEM((CH,), jnp.int32),
            pltpu.VMEM((CH,), jnp.int32),
            pltpu.VMEM((CH,), jnp.int32),
            pltpu.VMEM((CH,), jnp.int32),
            pltpu.VMEM((CH, H), jnp.float32),
            pltpu.VMEM((CH, H), jnp.float32),
            pltpu.VMEM((CH, H), jnp.float32),
            pltpu.VMEM((CH, H), jnp.float32),
            pltpu.VMEM_SHARED((NACC, H), jnp.float32),
            pltpu.SemaphoreType.DMA,
            pltpu.SemaphoreType.DMA,
            pltpu.SemaphoreType.DMA,
            pltpu.SemaphoreType.DMA,
            pltpu.SemaphoreType.DMA,
            pltpu.SemaphoreType.DMA,
            pltpu.SemaphoreType.DMA,
            pltpu.SemaphoreType.DMA,
            pltpu.SemaphoreType.DMA,
        ],
    )


def _sc_body(t_hbm, src_hbm, et_hbm, dst_hbm, out_hbm,
             src_q, et_q, dst_q, gi0, gi1, gi2, gi3, dc0, dc1, dc2, dc3,
             rw0, rw1, rw2, rw3, accum, semi,
             sg0, sg1, sg2, sg3, ss0, ss1, ss2, ss3):
    gidx = (gi0, gi1, gi2, gi3)
    dstc = (dc0, dc1, dc2, dc3)
    rows = (rw0, rw1, rw2, rw3)
    semg = (sg0, sg1, sg2, sg3)
    sems = (ss0, ss1, ss2, ss3)
    c = lax.axis_index("c")
    s = lax.axis_index("s")
    zeros16 = jnp.zeros((16,), jnp.float32)
    base_row = pl.multiple_of(s * ROWS_TILE, 8)
    nq = jnp.where(c == 0, Q0, Q1)
    tile_base = pl.multiple_of(
        c * (NTILE * Q0 * QCH) + s * nq * QCH, QCH)

    def _zrow(j, _):
        for k in range(H // 16):
            rw0[j, pl.ds(k * 16, 16)] = zeros16
        return 0
    lax.fori_loop(0, CH, _zrow, 0)
    zspans = [(k * CH, CH) for k in range(ROWS_TILE // CH)]
    zspans.append((ROWS_TILE // CH * CH, ROWS_TILE % CH))
    for r0, ln in zspans:
        pltpu.sync_copy(rw0.at[pl.ds(0, ln)],
                        accum.at[pl.ds(base_row + r0, ln)])
    plsc.subcore_barrier()

    def _quad(m, _):
        off = pl.multiple_of(tile_base + m * QCH, QCH)
        lq_s = pltpu.async_copy(src_hbm.at[pl.ds(off, QCH)], src_q, semi)
        lq_e = pltpu.async_copy(et_hbm.at[pl.ds(off, QCH)], et_q, semi)
        lq_d = pltpu.async_copy(dst_hbm.at[pl.ds(off, QCH)], dst_q, semi)
        lq_s.wait()
        lq_e.wait()
        lq_d.wait()
        gs = []
        for j in range(NBUF):
            def _gix(i, _, j=j):
                sl = pl.ds(pl.multiple_of(i * 16, 16), 16)
                qsl = pl.ds(pl.multiple_of(j * CH + i * 16, 16), 16)
                gidx[j][sl] = et_q[qsl] * NP + src_q[qsl]
                dstc[j][sl] = dst_q[qsl]
                return 0
            lax.fori_loop(0, CH // 16, _gix, 0)
            gs.append(pltpu.async_copy(t_hbm.at[gidx[j]], rows[j], semg[j]))
        sc = []
        for j in range(NBUF):
            gs[j].wait()
            sc.append(pltpu.async_copy(rows[j], accum.at[dstc[j]],
                                       sems[j], add=True))
        for j in range(NBUF):
            sc[j].wait()
        return 0
    lax.fori_loop(0, nq, _quad, 0)
    plsc.subcore_barrier()

    for i, (r0, ln) in enumerate(zspans):
        buf = rows[i % NBUF]
        rr = base_row + r0
        pltpu.sync_copy(accum.at[pl.ds(rr, ln)], buf.at[pl.ds(0, ln)])
        pltpu.sync_copy(buf.at[pl.ds(0, ln)], out_hbm.at[c, pl.ds(rr, ln)])


def kernel(class_objects, states_objects, edge_tuples, edge_classes,
           mask_object, mask_edge, class_table, Ws, bs, W1, b1, W2, b2,
           Wl, bl, W_ih, b_ih, W_hh, b_hh):
    f32 = jnp.float32
    num_envs = class_objects.shape[0]

    ct_p = jnp.zeros((H, 64), f32).at[:class_table.shape[0]].set(class_table)
    ws_p = jnp.zeros((64, H), f32).at[:, :Ws.shape[1]].set(Ws)
    w1a = W1[:, :64]
    w1b = W1[:, 64:]
    bs2 = bs.reshape(1, 64)
    b12 = b1.reshape(1, H)
    b22 = b2.reshape(1, H)
    bih2 = b_ih.reshape(1, 3 * H)
    bhh2 = b_hh.reshape(1, 3 * H)

    epad = EPAD - E
    pad_dst = (jnp.arange(epad, dtype=jnp.int32) % N)

    hs, ts, edges = [], [], []
    for env in range(num_envs):
        ids = class_objects[env].astype(jnp.int32)
        ids_p = jnp.zeros((NP,), jnp.int32).at[:N].set(ids)
        idsb = jnp.broadcast_to(ids_p[:, None], (NP, H))
        states_p = jnp.zeros((NP, H), f32).at[:N, :states_objects.shape[2]].set(
            states_objects[env])

        src = edge_tuples[env, :, 0].astype(jnp.int32)
        dst = edge_tuples[env, :, 1].astype(jnp.int32)
        et = edge_classes[env].astype(jnp.int32)
        edges.append((
            jnp.concatenate([src, jnp.zeros((epad,), jnp.int32)]),
            jnp.concatenate([et, jnp.zeros((epad,), jnp.int32)]),
            jnp.concatenate([dst, pad_dst]),
        ))
        h, t = _encoder(idsb, states_p, ct_p, ws_p, bs2, w1a, w1b, b12,
                        W2, b22, Wl, bl)
        hs.append(h)
        ts.append(t)

    for step in range(K):
        aps = []
        for env in range(num_envs):
            src_p, et_p, dst_p = edges[env]
            ap = _sc_kernel()(ts[env].reshape((R + 1) * NP, H),
                              src_p, et_p, dst_p)
            aps.append(jnp.pad(ap, ((0, 0), (0, NP - NACC), (0, 0))))
        for env in range(num_envs):
            if step < K - 1:
                hs[env], ts[env] = _gru(True, aps[env], hs[env], W_ih, bih2,
                                        W_hh, bhh2, Wl, bl)
            else:
                (hs[env],) = _gru(False, aps[env], hs[env], W_ih, bih2,
                                  W_hh, bhh2, Wl, bl)
    return jnp.stack([h[:N] for h in hs], axis=0)
